# baseline (speedup 1.0000x reference)
import functools

import jax
import jax.numpy as jnp
from jax import lax
from jax.experimental import pallas as pl
from jax.experimental.pallas import tpu as pltpu
from jax.experimental.pallas import tpu_sc as plsc

D = 128
NC, NS = 2, 16
NW = NC * NS
CH = 160
LAG = 2


@functools.lru_cache(maxsize=None)
def _make_gather(b: int, v: int):
    bpw = -(-b // NW)
    bpw = ((bpw + 7) // 8) * 8
    assert b >= bpw >= CH and b % 8 == 0
    nch = -(-bpw // CH)
    nbuf = next((k for k in (5, 4, 6, 3, 7, 2) if nch % k == 0), None)
    if nbuf is None:
        nbuf = 4
        nch = -(-nch // nbuf) * nbuf
    nout = nch // nbuf
    mesh = plsc.VectorSubcoreMesh(core_axis_name="c", subcore_axis_name="s")

    @functools.partial(
        pl.kernel,
        out_type=jax.ShapeDtypeStruct((b, D), jnp.float32),
        mesh=mesh,
        scratch_types=[
            pltpu.VMEM((bpw,), jnp.int32),
            pltpu.VMEM((nbuf, CH, D), jnp.float32),
            [pltpu.SemaphoreType.DMA] * nbuf,
            [pltpu.SemaphoreType.DMA] * nbuf,
        ],
    )
    def gather_kernel(x_hbm, idx_hbm, out_hbm, idx_v, buf_v, gsems, wsems):
        wid = lax.axis_index("s") * NC + lax.axis_index("c")
        base = wid * bpw
        stage = jnp.minimum(base, b - bpw)
        end = jnp.minimum(base + bpw, b)
        pltpu.sync_copy(idx_hbm.at[pl.ds(stage, bpw)], idx_v)

        def start(g):
            return jnp.minimum(base + g * CH, end - CH)

        def gather(g, bf):
            s = start(g)
            pltpu.async_copy(
                x_hbm.at[idx_v.at[pl.ds(s - stage, CH)]], buf_v.at[bf],
                gsems[bf])

        def wait_gather(bf):
            pltpu.make_async_copy(
                x_hbm.at[idx_v.at[pl.ds(0, CH)]], buf_v.at[bf], gsems[bf]
            ).wait()

        def write(g, bf):
            pltpu.async_copy(
                buf_v.at[bf], out_hbm.at[pl.ds(start(g), CH)], wsems[bf])

        def wait_write(bf):
            pltpu.make_async_copy(
                buf_v.at[bf], out_hbm.at[pl.ds(0, CH)], wsems[bf]).wait()

        for bf in range(LAG):
            gather(bf, bf)

        @pl.loop(0, nout)
        def _(j):
            for bf in range(nbuf):
                g = j * nbuf + bf
                wait_gather(bf)
                write(g, bf)
                bb = (bf + LAG) % nbuf
                gg = g + LAG

                @pl.when(gg >= nbuf)
                def _():
                    wait_write(bb)

                @pl.when(gg < nch)
                def _():
                    gather(gg, bb)

        for bf in range(LAG, nbuf):
            wait_write(bf)

    return gather_kernel


def kernel(x, upsamples):
    idx = upsamples.reshape(-1)
    return _make_gather(idx.shape[0], x.shape[0])(x, idx)


# speedup vs baseline: 1.0141x; 1.0141x over previous
import functools

import jax
import jax.numpy as jnp
from jax import lax
from jax.experimental import pallas as pl
from jax.experimental.pallas import tpu as pltpu
from jax.experimental.pallas import tpu_sc as plsc

D = 128
NC, NS = 2, 16
NW = NC * NS
CH = 128
LAG = 3


@functools.lru_cache(maxsize=None)
def _make_gather(b: int, v: int):
    bpw = -(-b // NW)
    bpw = ((bpw + 7) // 8) * 8
    assert b >= bpw >= CH and b % 8 == 0
    nch = -(-bpw // CH)
    nbuf = next((k for k in (5, 4, 6, 3, 7, 2) if nch % k == 0), None)
    if nbuf is None:
        nbuf = 4
        nch = -(-nch // nbuf) * nbuf
    nout = nch // nbuf
    mesh = plsc.VectorSubcoreMesh(core_axis_name="c", subcore_axis_name="s")

    @functools.partial(
        pl.kernel,
        out_type=jax.ShapeDtypeStruct((b, D), jnp.float32),
        mesh=mesh,
        scratch_types=[
            pltpu.VMEM((bpw,), jnp.int32),
            pltpu.VMEM((nbuf, CH, D), jnp.float32),
            [pltpu.SemaphoreType.DMA] * nbuf,
            [pltpu.SemaphoreType.DMA] * nbuf,
        ],
    )
    def gather_kernel(x_hbm, idx_hbm, out_hbm, idx_v, buf_v, gsems, wsems):
        wid = lax.axis_index("s") * NC + lax.axis_index("c")
        base = wid * bpw
        stage = jnp.minimum(base, b - bpw)
        end = jnp.minimum(base + bpw, b)
        pltpu.sync_copy(idx_hbm.at[pl.ds(stage, bpw)], idx_v)

        def start(g):
            return jnp.minimum(base + g * CH, end - CH)

        def gather(g, bf):
            s = start(g)
            pltpu.async_copy(
                x_hbm.at[idx_v.at[pl.ds(s - stage, CH)]], buf_v.at[bf],
                gsems[bf])

        def wait_gather(bf):
            pltpu.make_async_copy(
                x_hbm.at[idx_v.at[pl.ds(0, CH)]], buf_v.at[bf], gsems[bf]
            ).wait()

        def write(g, bf):
            pltpu.async_copy(
                buf_v.at[bf], out_hbm.at[pl.ds(start(g), CH)], wsems[bf])

        def wait_write(bf):
            pltpu.make_async_copy(
                buf_v.at[bf], out_hbm.at[pl.ds(0, CH)], wsems[bf]).wait()

        for bf in range(LAG):
            gather(bf, bf)

        @pl.loop(0, nout)
        def _(j):
            for bf in range(nbuf):
                g = j * nbuf + bf
                wait_gather(bf)
                write(g, bf)
                bb = (bf + LAG) % nbuf
                gg = g + LAG

                @pl.when(gg >= nbuf)
                def _():
                    wait_write(bb)

                @pl.when(gg < nch)
                def _():
                    gather(gg, bb)

        for bf in range(LAG, nbuf):
            wait_write(bf)

    return gather_kernel


def kernel(x, upsamples):
    idx = upsamples.reshape(-1)
    return _make_gather(idx.shape[0], x.shape[0])(x, idx)
